# baseline (speedup 1.0000x reference)
import functools

import jax
import jax.numpy as jnp
from jax import lax
from jax.experimental import pallas as pl
from jax.experimental.pallas import tpu as pltpu
from jax.experimental.pallas import tpu_sc as plsc

_TN = 1024
_TNC = 1024
_CHUNK = 128

_BIG = 1e30


def _nn_body(u_ref, kT_ref, kf_ref, idx0_ref, idx1_ref, idx2_ref,
             w0_ref, w1_ref, w2_ref, tbl_ref, *, m):
    b = pl.program_id(0)
    tbl_ref[...] = jnp.transpose(kf_ref[...], (1, 0))

    u = u_ref[...]
    kT = kT_ref[...]
    ux, uy, uz = u[:, 0:1], u[:, 1:2], u[:, 2:3]
    kx, ky, kz = kT[0:1, :], kT[1:2, :], kT[2:3, :]
    d2 = (ux - kx) ** 2 + (uy - ky) ** 2 + (uz - kz) ** 2

    iota = lax.broadcasted_iota(jnp.int32, d2.shape, 1)
    D = d2
    mins, idxs = [], []
    for s in range(3):
        mn = jnp.min(D, axis=1, keepdims=True)
        cand = jnp.where(D == mn, iota, m)
        amn = jnp.min(cand, axis=1, keepdims=True)
        mins.append(mn)
        idxs.append(amn)
        if s < 2:
            D = jnp.where(cand == amn, _BIG, D)

    mns = jnp.concatenate(mins, axis=1)
    wsv = 1.0 / (jnp.sqrt(mns) + 1e-8)
    wn = wsv / jnp.sum(wsv, axis=1, keepdims=True)
    wT = jnp.transpose(wn, (1, 0))
    base = b * m
    for idx, iref in zip(idxs, (idx0_ref, idx1_ref, idx2_ref)):
        iref[...] = jnp.reshape(idx + base, (_TN // 128, 128))
    for s, wref in enumerate((w0_ref, w1_ref, w2_ref)):
        wref[...] = wT[s:s + 1, :]


def _three_nn(unknown, knownT, known_feats):
    B, n, _ = unknown.shape
    m = knownT.shape[2]
    C2 = known_feats.shape[1]
    nblk = n // _TN
    msub = 256
    ntb = m // msub
    grid = (B, nblk)
    iout = jax.ShapeDtypeStruct((B * n // 128, 128), jnp.int32)
    fout = jax.ShapeDtypeStruct((B * nblk, 1, _TN), jnp.float32)
    ispec = pl.BlockSpec((_TN // 128, 128), lambda b, i: (b * nblk + i, 0))
    wspec = pl.BlockSpec((None, 1, _TN), lambda b, i: (b * nblk + i, 0, 0))
    return pl.pallas_call(
        functools.partial(_nn_body, m=m),
        grid=grid,
        in_specs=[
            pl.BlockSpec((None, _TN, 3), lambda b, i: (b, i, 0)),
            pl.BlockSpec((None, 3, m), lambda b, i: (b, 0, 0)),
            pl.BlockSpec((None, C2, msub), lambda b, i: (b, 0, i % ntb)),
        ],
        out_specs=[ispec, ispec, ispec, wspec, wspec, wspec,
                   pl.BlockSpec((msub, C2),
                                lambda b, i: (b * ntb + i % ntb, 0))],
        out_shape=[iout, iout, iout, fout, fout, fout,
                   jax.ShapeDtypeStruct((B * m, C2), jnp.float32)],
    )(unknown, knownT, known_feats)


def _sc_gather(table, idx0, idx1, idx2):
    nrows, chunk = idx0.shape
    N = nrows * chunk
    C2 = table.shape[1]
    info = plsc.get_sparse_core_info()
    nw = info.num_cores * info.num_subcores
    per_w = N // nw
    nchunk = per_w // chunk
    ntask = 3 * nchunk
    nbuf = 4
    mesh = plsc.VectorSubcoreMesh(core_axis_name="c", subcore_axis_name="s")
    gout = jax.ShapeDtypeStruct((N, C2), jnp.float32)

    @functools.partial(
        pl.kernel, mesh=mesh,
        out_type=(gout, gout, gout),
        scratch_types=[
            pltpu.VMEM((ntask, chunk), jnp.int32),
            pltpu.VMEM((nbuf, chunk, C2), jnp.float32),
            [pltpu.SemaphoreType.DMA] * nbuf,
            [pltpu.SemaphoreType.DMA] * nbuf,
        ],
    )
    def gather_kernel(table_hbm, i0_hbm, i1_hbm, i2_hbm,
                      g0_hbm, g1_hbm, g2_hbm, idx_all, rows, gsems, wsems):
        wid = lax.axis_index("s") * info.num_cores + lax.axis_index("c")
        row0 = wid * nchunk
        for j, ih in enumerate((i0_hbm, i1_hbm, i2_hbm)):
            pltpu.sync_copy(ih.at[pl.ds(row0, nchunk), :],
                            idx_all.at[pl.ds(j * nchunk, nchunk), :])

        ghandles = [None] * nbuf

        def start_gather(t):
            buf = t % nbuf
            ghandles[buf] = pltpu.async_copy(
                table_hbm.at[idx_all.at[t]], rows.at[buf], gsems[buf])

        for t in range(nbuf):
            start_gather(t)
        gouts = (g0_hbm, g1_hbm, g2_hbm)
        for t in range(ntask):
            buf = t % nbuf
            ghandles[buf].wait()
            j, c = divmod(t, nchunk)
            off = wid * per_w + c * chunk
            wh = pltpu.async_copy(rows.at[buf],
                                  gouts[j].at[pl.ds(off, chunk), :],
                                  wsems[buf])
            wh.wait()
            if t + nbuf < ntask:
                start_gather(t + nbuf)

    return gather_kernel(table, idx0, idx1, idx2)


def _mlp_body(g0_ref, g1_ref, g2_ref, w0_ref, w1_ref, w2_ref,
              uf_ref, W1_ref, b1_ref, out_ref, *, C2):
    wcat = jnp.concatenate(
        [w0_ref[...], w1_ref[...], w2_ref[...]], axis=0)
    wt = jnp.transpose(wcat, (1, 0))
    interp = (wt[:, 0:1] * g0_ref[...] + wt[:, 1:2] * g1_ref[...]
              + wt[:, 2:3] * g2_ref[...])
    W1 = W1_ref[...]
    acc = lax.dot_general(W1[:, :C2], interp,
                          (((1,), (1,)), ((), ())),
                          preferred_element_type=jnp.float32)
    acc = acc + jnp.dot(W1[:, C2:], uf_ref[...],
                        preferred_element_type=jnp.float32)
    out_ref[...] = jnp.maximum(acc + jnp.transpose(b1_ref[...], (1, 0)), 0.0)


def _mlp(g0, g1, g2, w0, w1, w2, unknow_feats, W1, b1r):
    B, C1, n = unknow_feats.shape
    C2 = g0.shape[1]
    Co = W1.shape[0]
    nblk = n // _TNC
    grid = (B, nblk)
    gspec = pl.BlockSpec((_TNC, C2), lambda b, i: (b * nblk + i, 0))
    wspec = pl.BlockSpec((None, 1, _TNC), lambda b, i: (b * nblk + i, 0, 0))
    return pl.pallas_call(
        functools.partial(_mlp_body, C2=C2),
        grid=grid,
        in_specs=[
            gspec, gspec, gspec, wspec, wspec, wspec,
            pl.BlockSpec((None, C1, _TNC), lambda b, i: (b, 0, i)),
            pl.BlockSpec((Co, W1.shape[1]), lambda b, i: (0, 0)),
            pl.BlockSpec((1, Co), lambda b, i: (0, 0)),
        ],
        out_specs=pl.BlockSpec((None, Co, _TNC), lambda b, i: (b, 0, i)),
        out_shape=jax.ShapeDtypeStruct((B, Co, n), jnp.float32),
    )(g0, g1, g2, w0, w1, w2, unknow_feats, W1, b1r)


def kernel(unknown, known, unknow_feats, known_feats, W1, b1):
    B, n, _ = unknown.shape
    knownT = jnp.transpose(known, (0, 2, 1))
    b1r = b1.reshape(1, -1)

    Bh = B // 2
    nns = []
    for h in range(2):
        s = slice(h * Bh, (h + 1) * Bh)
        nns.append(_three_nn(unknown[s], knownT[s], known_feats[s]))
    gs = []
    for h in range(2):
        idx0, idx1, idx2, w0, w1, w2, table = nns[h]
        gs.append(_sc_gather(table, idx0, idx1, idx2))
    outs = []
    for h in range(2):
        s = slice(h * Bh, (h + 1) * Bh)
        idx0, idx1, idx2, w0, w1, w2, table = nns[h]
        g0, g1, g2 = gs[h]
        outs.append(_mlp(g0, g1, g2, w0, w1, w2,
                         unknow_feats[s], W1, b1r))
    return jnp.concatenate(outs, axis=0)


# speedup vs baseline: 1.0084x; 1.0084x over previous
import functools

import jax
import jax.numpy as jnp
from jax import lax
from jax.experimental import pallas as pl
from jax.experimental.pallas import tpu as pltpu
from jax.experimental.pallas import tpu_sc as plsc

_TN = 1024
_TNC = 1024
_CHUNK = 128

_BIG = 1e30


def _nn_body(u_ref, kT_ref, kf_ref, idx0_ref, idx1_ref, idx2_ref,
             w0_ref, w1_ref, w2_ref, tbl_ref, *, m):
    b = pl.program_id(0)
    tbl_ref[...] = jnp.transpose(kf_ref[...], (1, 0))

    u = u_ref[...]
    kT = kT_ref[...]
    ux, uy, uz = u[:, 0:1], u[:, 1:2], u[:, 2:3]
    kx, ky, kz = kT[0:1, :], kT[1:2, :], kT[2:3, :]
    d2 = (ux - kx) ** 2 + (uy - ky) ** 2 + (uz - kz) ** 2

    iota = lax.broadcasted_iota(jnp.int32, d2.shape, 1)
    D = d2
    mins, idxs = [], []
    for s in range(3):
        mn = jnp.min(D, axis=1, keepdims=True)
        cand = jnp.where(D == mn, iota, m)
        amn = jnp.min(cand, axis=1, keepdims=True)
        mins.append(mn)
        idxs.append(amn)
        if s < 2:
            D = jnp.where(cand == amn, _BIG, D)

    mns = jnp.concatenate(mins, axis=1)
    wsv = 1.0 / (jnp.sqrt(mns) + 1e-8)
    wn = wsv / jnp.sum(wsv, axis=1, keepdims=True)
    wT = jnp.transpose(wn, (1, 0))
    base = b * m
    for idx, iref in zip(idxs, (idx0_ref, idx1_ref, idx2_ref)):
        iref[...] = jnp.reshape(idx + base, (_TN // 128, 128))
    for s, wref in enumerate((w0_ref, w1_ref, w2_ref)):
        wref[...] = wT[s:s + 1, :]


def _three_nn(unknown, knownT, known_feats):
    B, n, _ = unknown.shape
    m = knownT.shape[2]
    C2 = known_feats.shape[1]
    nblk = n // _TN
    msub = 256
    ntb = m // msub
    grid = (B, nblk)
    iout = jax.ShapeDtypeStruct((B * n // 128, 128), jnp.int32)
    fout = jax.ShapeDtypeStruct((B * nblk, 1, _TN), jnp.float32)
    ispec = pl.BlockSpec((_TN // 128, 128), lambda b, i: (b * nblk + i, 0))
    wspec = pl.BlockSpec((None, 1, _TN), lambda b, i: (b * nblk + i, 0, 0))
    return pl.pallas_call(
        functools.partial(_nn_body, m=m),
        grid=grid,
        in_specs=[
            pl.BlockSpec((None, _TN, 3), lambda b, i: (b, i, 0)),
            pl.BlockSpec((None, 3, m), lambda b, i: (b, 0, 0)),
            pl.BlockSpec((None, C2, msub), lambda b, i: (b, 0, i % ntb)),
        ],
        out_specs=[ispec, ispec, ispec, wspec, wspec, wspec,
                   pl.BlockSpec((msub, C2),
                                lambda b, i: (b * ntb + i % ntb, 0))],
        out_shape=[iout, iout, iout, fout, fout, fout,
                   jax.ShapeDtypeStruct((B * m, C2), jnp.float32)],
    )(unknown, knownT, known_feats)


def _sc_gather(table, idx0, idx1, idx2):
    nrows, chunk = idx0.shape
    N = nrows * chunk
    C2 = table.shape[1]
    info = plsc.get_sparse_core_info()
    nw = info.num_cores * info.num_subcores
    per_w = N // nw
    nchunk = per_w // chunk
    ntask = 3 * nchunk
    nbuf = 4
    mesh = plsc.VectorSubcoreMesh(core_axis_name="c", subcore_axis_name="s")
    gout = jax.ShapeDtypeStruct((N, C2), jnp.float32)

    @functools.partial(
        pl.kernel, mesh=mesh,
        out_type=(gout, gout, gout),
        scratch_types=[
            pltpu.VMEM((ntask, chunk), jnp.int32),
            pltpu.VMEM((nbuf, chunk, C2), jnp.float32),
            [pltpu.SemaphoreType.DMA] * nbuf,
            [pltpu.SemaphoreType.DMA] * nbuf,
        ],
    )
    def gather_kernel(table_hbm, i0_hbm, i1_hbm, i2_hbm,
                      g0_hbm, g1_hbm, g2_hbm, idx_all, rows, gsems, wsems):
        wid = lax.axis_index("s") * info.num_cores + lax.axis_index("c")
        row0 = wid * nchunk
        for j, ih in enumerate((i0_hbm, i1_hbm, i2_hbm)):
            pltpu.sync_copy(ih.at[pl.ds(row0, nchunk), :],
                            idx_all.at[pl.ds(j * nchunk, nchunk), :])

        ghandles = [None] * nbuf

        def start_gather(t):
            buf = t % nbuf
            ghandles[buf] = pltpu.async_copy(
                table_hbm.at[idx_all.at[t]], rows.at[buf], gsems[buf])

        for t in range(nbuf):
            start_gather(t)
        gouts = (g0_hbm, g1_hbm, g2_hbm)
        for t in range(ntask):
            buf = t % nbuf
            ghandles[buf].wait()
            j, c = divmod(t, nchunk)
            off = wid * per_w + c * chunk
            wh = pltpu.async_copy(rows.at[buf],
                                  gouts[j].at[pl.ds(off, chunk), :],
                                  wsems[buf])
            wh.wait()
            if t + nbuf < ntask:
                start_gather(t + nbuf)

    return gather_kernel(table, idx0, idx1, idx2)


def _mlp_body(g0_ref, g1_ref, g2_ref, w0_ref, w1_ref, w2_ref,
              uf_ref, W1_ref, b1_ref, out_ref, *, C2):
    wcat = jnp.concatenate(
        [w0_ref[...], w1_ref[...], w2_ref[...]], axis=0)
    wt = jnp.transpose(wcat, (1, 0))
    interp = (wt[:, 0:1] * g0_ref[...] + wt[:, 1:2] * g1_ref[...]
              + wt[:, 2:3] * g2_ref[...])
    W1 = W1_ref[...]
    acc = lax.dot_general(W1[:, :C2], interp,
                          (((1,), (1,)), ((), ())),
                          preferred_element_type=jnp.float32)
    acc = acc + jnp.dot(W1[:, C2:], uf_ref[...],
                        preferred_element_type=jnp.float32)
    out_ref[...] = jnp.maximum(acc + jnp.transpose(b1_ref[...], (1, 0)), 0.0)


def _mlp(g0, g1, g2, w0, w1, w2, unknow_feats, W1, b1r):
    B, C1, n = unknow_feats.shape
    C2 = g0.shape[1]
    Co = W1.shape[0]
    nblk = n // _TNC
    grid = (B, nblk)
    gspec = pl.BlockSpec((_TNC, C2), lambda b, i: (b * nblk + i, 0))
    wspec = pl.BlockSpec((None, 1, _TNC), lambda b, i: (b * nblk + i, 0, 0))
    return pl.pallas_call(
        functools.partial(_mlp_body, C2=C2),
        grid=grid,
        in_specs=[
            gspec, gspec, gspec, wspec, wspec, wspec,
            pl.BlockSpec((None, C1, _TNC), lambda b, i: (b, 0, i)),
            pl.BlockSpec((Co, W1.shape[1]), lambda b, i: (0, 0)),
            pl.BlockSpec((1, Co), lambda b, i: (0, 0)),
        ],
        out_specs=pl.BlockSpec((None, Co, _TNC), lambda b, i: (b, 0, i)),
        out_shape=jax.ShapeDtypeStruct((B, Co, n), jnp.float32),
    )(g0, g1, g2, w0, w1, w2, unknow_feats, W1, b1r)


def kernel(unknown, known, unknow_feats, known_feats, W1, b1):
    B, n, _ = unknown.shape

    knownT = jnp.transpose(known, (0, 2, 1))

    (idx0, idx1, idx2, w0, w1, w2, table) = _three_nn(
        unknown, knownT, known_feats)

    g0, g1, g2 = _sc_gather(table, idx0, idx1, idx2)

    b1r = b1.reshape(1, -1)
    return _mlp(g0, g1, g2, w0, w1, w2, unknow_feats, W1, b1r)
